# baseline (speedup 1.0000x reference)
import functools

import jax
import jax.numpy as jnp
from jax import lax
from jax.experimental import pallas as pl
from jax.experimental.pallas import tpu as pltpu
from jax.experimental.pallas import tpu_sc as plsc

_D = 64
_BETA = 0.25
_BM = 256
_BN = 512
_TILE = 4096


def _vq_argmin_body(zsq_ref, esq_ref, z_ref, emb_ref, idx_ref, loss_ref,
                    bval_ref, bidx_ref, cval_ref, cidx_ref, craw_ref):
    i = pl.program_id(0)
    j = pl.program_id(1)
    nj = pl.num_programs(1)
    jt = _TILE // _BN

    @pl.when(j % jt == 0)
    def _init():
        bval_ref[...] = jnp.full_like(bval_ref, jnp.inf)
        bidx_ref[...] = jnp.zeros_like(bidx_ref)

    zt = z_ref[...]
    zsq = zsq_ref[...]
    esq = esq_ref[...]
    lane = lax.broadcasted_iota(jnp.int32, (zt.shape[0], 128), 1).astype(
        jnp.float32)
    bv = bval_ref[...]
    bi = bidx_ref[...]
    for c in range(_BN // 128):
        dot = lax.dot_general(zt, emb_ref[pl.ds(c * 128, 128), :],
                              (((1,), (1,)), ((), ())),
                              preferred_element_type=jnp.float32)
        cand = (zsq + esq[:, c * 128:(c + 1) * 128]) - 2.0 * dot
        ci = lane + (j * _BN + c * 128).astype(jnp.float32)
        better = cand < bv
        bi = jnp.where(better, ci, bi)
        bv = jnp.where(better, cand, bv)
    bval_ref[...] = bv
    bidx_ref[...] = bi

    @pl.when(j % jt == jt - 1)
    def _tile_end():
        gm = jnp.min(bv, axis=1, keepdims=True)
        gi = jnp.min(jnp.where(bv == gm, bi, jnp.float32(1e9)),
                     axis=1, keepdims=True)
        gmr = gm.astype(jnp.bfloat16).astype(jnp.float32)

        @pl.when(j == jt - 1)
        def _first_tile():
            cval_ref[...] = gmr
            cidx_ref[...] = gi
            craw_ref[...] = gm

        @pl.when(j > jt - 1)
        def _combine():
            take = gm < cval_ref[...]
            cval_ref[...] = jnp.where(take, gmr, cval_ref[...])
            cidx_ref[...] = jnp.where(take, gi, cidx_ref[...])
            craw_ref[...] = jnp.where(take, gm, craw_ref[...])

    @pl.when(j == nj - 1)
    def _finish():
        idx_ref[...] = cidx_ref[...].astype(jnp.int32)
        ssum = jnp.sum(craw_ref[...]).reshape(1, 1)

        @pl.when(i == 0)
        def _first():
            loss_ref[...] = ssum

        @pl.when(i > 0)
        def _rest():
            loss_ref[...] = loss_ref[...] + ssum


def _argmin_tc(z_flat, emb_weight, zsq, esq):
    m = z_flat.shape[0]
    n = emb_weight.shape[0]
    grid = (m // _BM, n // _BN)
    return pl.pallas_call(
        _vq_argmin_body,
        grid=grid,
        in_specs=[
            pl.BlockSpec((_BM, 1), lambda i, j: (i, 0)),
            pl.BlockSpec((1, _BN), lambda i, j: (0, j)),
            pl.BlockSpec((_BM, _D), lambda i, j: (i, 0)),
            pl.BlockSpec((_BN, _D), lambda i, j: (j, 0)),
        ],
        out_specs=[
            pl.BlockSpec((_BM, 1), lambda i, j: (i, 0)),
            pl.BlockSpec((1, 1), lambda i, j: (0, 0)),
        ],
        out_shape=[
            jax.ShapeDtypeStruct((m, 1), jnp.int32),
            jax.ShapeDtypeStruct((1, 1), jnp.float32),
        ],
        scratch_shapes=[
            pltpu.VMEM((_BM, 128), jnp.float32),
            pltpu.VMEM((_BM, 128), jnp.float32),
            pltpu.VMEM((_BM, 1), jnp.float32),
            pltpu.VMEM((_BM, 1), jnp.float32),
            pltpu.VMEM((_BM, 1), jnp.float32),
        ],
        compiler_params=pltpu.CompilerParams(
            dimension_semantics=("arbitrary", "arbitrary")),
    )(zsq, esq, z_flat, emb_weight)


def _gather_sc(table_pad, idx_flat):
    m = idx_flat.shape[0]
    dp = table_pad.shape[1]
    nc, ns = 2, 16
    nw = nc * ns
    bpw = m // nw
    mesh = plsc.VectorSubcoreMesh(core_axis_name="c", subcore_axis_name="s")

    @functools.partial(
        pl.kernel,
        mesh=mesh,
        out_type=jax.ShapeDtypeStruct((m, dp), jnp.float32),
        scratch_types=[
            pltpu.VMEM((bpw,), jnp.int32),
            pltpu.VMEM((bpw, dp), jnp.float32),
            pltpu.SemaphoreType.DMA,
        ],
    )
    def gk(table_hbm, idx_hbm, out_hbm, idx_v, rows_v, sem):
        wid = lax.axis_index("s") * nc + lax.axis_index("c")
        base = wid * bpw
        pltpu.sync_copy(idx_hbm.at[pl.ds(base, bpw)], idx_v)
        pltpu.async_copy(table_hbm.at[idx_v], rows_v, sem).wait()
        pltpu.sync_copy(rows_v, out_hbm.at[pl.ds(base, bpw)])

    return gk(table_pad, idx_flat)


def kernel(z, emb_weight):
    z = z.astype(jnp.float32)
    z_p = jnp.transpose(z, (0, 2, 3, 1))
    z_flat = z_p.reshape(-1, emb_weight.shape[1])
    zsq = jnp.sum(z_flat ** 2, axis=1, keepdims=True)
    esq = jnp.sum(emb_weight ** 2, axis=1).reshape(1, -1)

    idx2, loss_sum = _argmin_tc(z_flat, emb_weight, zsq, esq)
    min_indices = idx2.reshape(-1)

    table_pad = jnp.pad(emb_weight, ((0, 0), (0, 128 - _D)))
    z_q = _gather_sc(table_pad, min_indices)[:, :_D].reshape(z_p.shape)

    mean_sq = loss_sum[0, 0] / jnp.float32(z_flat.size)
    loss = mean_sq + _BETA * mean_sq

    z_q_st = z_p + lax.stop_gradient(z_q - z_p)
    z_q_out = jnp.transpose(z_q_st, (0, 3, 1, 2))
    return (z_q_out, loss, min_indices)
